# Initial kernel scaffold; baseline (speedup 1.0000x reference)
import jax
import jax.numpy as jnp
from jax.experimental import pallas as pl


def kernel(feature_map, A_bank, Wp, bp, Wproj, bproj, r, M, P):
    raise NotImplementedError("write your pallas kernel here")


# speedup vs baseline: 1.1495x; 1.1495x over previous
import functools

import jax
import jax.numpy as jnp
from jax import lax
from jax.experimental import pallas as pl
from jax.experimental.pallas import tpu as pltpu

_B = 2; _C = 96; _H = 64; _W = 64; _KP = 9; _STR = 2; _MCH = 64; _D = 128
_T = 16; _TOPK = 32; _SIGMA = 3.0; _ALPHA = 10.0; _TAU = 10.0; _EPS = 1e-06
_N = _H * _W
_K = (_H // _STR) * (_W // _STR)


def _embed_body(m_ref, g_ref, e_ref):
    m = m_ref[0]
    g = g_ref[0]
    e = jnp.dot(m.astype(jnp.bfloat16), g.astype(jnp.bfloat16),
                preferred_element_type=jnp.float32)
    nrm = jnp.sqrt(jnp.sum(e * e, axis=1, keepdims=True))
    e_ref[0] = e / jnp.maximum(nrm, _EPS)


def _embed(M, g, interpret=False):
    RT = 512
    grid = (_B, _N // RT)
    return pl.pallas_call(
        _embed_body,
        grid=grid,
        in_specs=[
            pl.BlockSpec((1, RT, _K), lambda b, i: (0, i, 0)),
            pl.BlockSpec((1, _K, _D), lambda b, i: (b, 0, 0)),
        ],
        out_specs=pl.BlockSpec((1, RT, _D), lambda b, i: (b, i, 0)),
        out_shape=jax.ShapeDtypeStruct((_B, _N, _D), jnp.float32),
        interpret=interpret,
    )(M[None], g)


def _csum_lanes(x):
    n = x.shape[1]
    sh = 1
    while sh < n:
        x = x + jnp.concatenate(
            [jnp.zeros_like(x[:, :sh]), x[:, :-sh]], axis=1)
        sh *= 2
    return x


def _affinity_body(er_ref, ef_ref, out_ref, s_scr):
    i = pl.program_id(1)
    RT = er_ref.shape[1]
    CB = 512

    e_rows = er_ref[0]
    row_ids = i * RT + lax.broadcasted_iota(jnp.int32, (RT, CB), 0)
    rx = (row_ids % _W).astype(jnp.float32)
    ry = (row_ids // _W).astype(jnp.float32)

    for ct in range(_N // CB):
        e_cols = ef_ref[0, pl.ds(ct * CB, CB), :]
        s = lax.dot_general(e_rows.astype(jnp.bfloat16),
                            e_cols.astype(jnp.bfloat16),
                            (((1,), (1,)), ((), ())),
                            preferred_element_type=jnp.float32)
        col_ids = ct * CB + lax.broadcasted_iota(jnp.int32, (RT, CB), 1)
        cx = (col_ids % _W).astype(jnp.float32)
        cy = (col_ids // _W).astype(jnp.float32)
        d2 = (rx - cx) ** 2 + (ry - cy) ** 2
        gmask = 1.0 - jnp.exp(d2 * (-1.0 / (2.0 * _SIGMA * _SIGMA)))
        s_scr[pl.ds(0, RT), pl.ds(ct * CB, CB)] = s * gmask

    GR = 8
    for gi in range(RT // GR):
        s0 = s_scr[pl.ds(gi * GR, GR), :]

        def step(_, carry):
            s, k, thr = carry
            m = jnp.max(s, axis=1, keepdims=True)
            hit = s == m
            cnt = jnp.sum(hit.astype(jnp.float32), axis=1, keepdims=True)
            live = k < float(_TOPK)
            s = jnp.where(live & hit, -3.0, s)
            thr = jnp.where(live, m, thr)
            k = k + jnp.where(live, cnt, 0.0)
            return s, k, thr

        _, _, thr = lax.fori_loop(
            0, _TOPK, step,
            (s0, jnp.zeros((GR, 1), jnp.float32),
             jnp.full((GR, 1), -3.0, jnp.float32)))
        gt = s0 > thr
        eq = s0 == thr
        need = (float(_TOPK)
                - jnp.sum(gt.astype(jnp.float32), axis=1, keepdims=True))
        csum = _csum_lanes(eq.astype(jnp.float32))
        sel = gt | (eq & (csum <= need))
        a = jnp.where(sel, jnp.exp(_ALPHA * s0), 0.0)
        denom = jnp.sum(a, axis=1, keepdims=True)
        out_ref[0, pl.ds(gi * GR, GR), :] = a / denom


def _affinity(E, interpret=False):
    RT = 256
    grid = (_B, _N // RT)
    return pl.pallas_call(
        _affinity_body,
        grid=grid,
        in_specs=[
            pl.BlockSpec((1, RT, _D), lambda b, i: (b, i, 0)),
            pl.BlockSpec((1, _N, _D), lambda b, i: (b, 0, 0)),
        ],
        out_specs=pl.BlockSpec((1, RT, _N), lambda b, i: (b, i, 0)),
        out_shape=jax.ShapeDtypeStruct((_B, _N, _N), jnp.float32),
        scratch_shapes=[pltpu.VMEM((RT, _N), jnp.float32)],
        interpret=interpret,
    )(E, E)


def _stage_a(feature_map, A_bank, Wp, bp, Wproj, bproj, r):
    b, c, h, w = feature_map.shape
    pad = _KP // 2
    xs = (2.0 * jnp.arange(w, dtype=jnp.float32) + 1.0) / w - 1.0
    ys = (2.0 * jnp.arange(h, dtype=jnp.float32) + 1.0) / h - 1.0
    gy, gx = jnp.meshgrid(ys, xs, indexing='ij')
    base = jnp.stack([gx, gy, jnp.ones_like(gx)], axis=-1)
    grid = jnp.einsum('hwk,tok->thwo', base, A_bank)

    ix = jnp.clip(((grid[..., 0] + 1.0) * w - 1.0) / 2.0, 0.0, w - 1.0)
    iy = jnp.clip(((grid[..., 1] + 1.0) * h - 1.0) / 2.0, 0.0, h - 1.0)
    x0 = jnp.floor(ix); y0 = jnp.floor(iy)
    wx = ix - x0; wy = iy - y0
    x0i = jnp.clip(x0, 0, w - 1).astype(jnp.int32)
    x1i = jnp.clip(x0 + 1, 0, w - 1).astype(jnp.int32)
    y0i = jnp.clip(y0, 0, h - 1).astype(jnp.int32)
    y1i = jnp.clip(y0 + 1, 0, h - 1).astype(jnp.int32)

    imp = jnp.transpose(feature_map, (0, 2, 3, 1))
    def warp_t(y0i, x0i, y1i, x1i, wx, wy):
        v00 = imp[:, y0i, x0i]; v01 = imp[:, y0i, x1i]
        v10 = imp[:, y1i, x0i]; v11 = imp[:, y1i, x1i]
        wx_ = wx[None, ..., None]; wy_ = wy[None, ..., None]
        return (v00 * (1 - wx_) * (1 - wy_) + v01 * wx_ * (1 - wy_)
                + v10 * (1 - wx_) * wy_ + v11 * wx_ * wy_)
    Fw = jax.vmap(warp_t, in_axes=(0, 0, 0, 0, 0, 0), out_axes=0)(
        y0i, x0i, y1i, x1i, wx, wy)
    Fw = jnp.transpose(Fw, (0, 1, 4, 2, 3)).reshape(_T * b, c, h, w)

    psum = lax.reduce_window(Fw, 0.0, lax.add, (1, 1, _KP, _KP),
                             (1, 1, _STR, _STR),
                             ((0, 0), (0, 0), (pad, pad), (pad, pad)))
    pm = psum / float(_KP * _KP)
    pm = jnp.transpose(pm.reshape(_T, b, c, _K), (0, 1, 3, 2))

    x = pm @ Wp.T + bp
    z = x @ Wproj.T + bproj
    z = z / jnp.maximum(jnp.linalg.norm(z, axis=-1, keepdims=True), _EPS)
    sc = jnp.sum(z * r, axis=-1)
    wgt = jax.nn.softmax(_TAU * sc, axis=0)[..., None]
    g = jnp.sum(wgt * z, axis=0)
    return g


def kernel(feature_map, A_bank, Wp, bp, Wproj, bproj, r, M, P):
    g = _stage_a(feature_map, A_bank, Wp, bp, Wproj, bproj, r)
    E = _embed(M, g)
    A0 = _affinity(E)
    return E, A0
